# baseline (speedup 1.0000x reference)
import functools

import jax
import jax.numpy as jnp
from jax import lax
from jax.experimental import pallas as pl
from jax.experimental.pallas import tpu as pltpu
from jax.experimental.pallas import tpu_sc as plsc

B, L, EMB = 16384, 200, 128
N = B * L
D = 4 * EMB
V = 4 * 2 * 2 * 6
NC, NS = 2, 16
NW = NC * NS
TOK_PER_W = N // NW
C = 64
NCHUNK = TOK_PER_W // C


def _build_fused_table(street, action, position, order):
    def body(s_ref, a_ref, p_ref, o_ref, out_ref):
        i = lax.broadcasted_iota(jnp.int32, (V, 1), 0)
        c0 = i // 24
        c1 = (i // 12) % 2
        c2 = (i // 6) % 2
        c3 = i % 6

        def sel(code, ref, nrows):
            out = jnp.broadcast_to(ref[nrows - 1:nrows, :], (V, EMB))
            for r in range(nrows - 2, -1, -1):
                row = jnp.broadcast_to(ref[r:r + 1, :], (V, EMB))
                out = jnp.where(code == r, row, out)
            return out

        out_ref[:, 0:EMB] = sel(c0, s_ref, 4)
        out_ref[:, EMB:2 * EMB] = sel(c1, p_ref, 2) + o_ref[0:1, :]
        out_ref[:, 2 * EMB:3 * EMB] = sel(c2, p_ref, 2) + o_ref[1:2, :]
        out_ref[:, 3 * EMB:4 * EMB] = sel(c3, a_ref, 6)

    return pl.pallas_call(
        body,
        out_shape=jax.ShapeDtypeStruct((V, D), jnp.float32),
    )(street, action, position, order)


def _sc_lookup(x_flat, fused):
    mesh = plsc.VectorSubcoreMesh(core_axis_name="c", subcore_axis_name="s")

    @functools.partial(
        pl.kernel,
        mesh=mesh,
        out_type=jax.ShapeDtypeStruct((N * D,), jnp.float32),
        compiler_params=pltpu.CompilerParams(needs_layout_passes=False),
        scratch_types=[
            pltpu.VMEM((C * 4,), jnp.int32),
            pltpu.VMEM((C,), jnp.int32),
            pltpu.VMEM((V * D,), jnp.float32),
            pltpu.VMEM((C * D,), jnp.float32),
            pltpu.VMEM((C * D,), jnp.float32),
            pltpu.SemaphoreType.DMA,
            pltpu.SemaphoreType.DMA,
        ],
    )
    def k(x_hbm, tab_hbm, out_hbm, x_v, ids_v, tab_v, buf0, buf1, sem0, sem1):
        cid = lax.axis_index("c")
        sid = lax.axis_index("s")
        wid = sid * NC + cid
        base = wid * TOK_PER_W

        iota = lax.iota(jnp.int32, 16)
        pltpu.sync_copy(tab_hbm, tab_v)

        def do_chunk(i, buf, sem):
            tok0 = base + i * C

            @pl.when(i >= 2)
            def _():
                pltpu.make_async_copy(
                    buf, out_hbm.at[pl.ds(tok0 * D, C * D)], sem).wait()

            pltpu.sync_copy(x_hbm.at[pl.ds(tok0 * 4, C * 4)], x_v)
            for g in range(C // 16):
                lanes = iota * 4 + g * 64
                v0 = plsc.load_gather(x_v, [lanes])
                v1 = plsc.load_gather(x_v, [lanes + 1])
                v2 = plsc.load_gather(x_v, [lanes + 2])
                v3 = plsc.load_gather(x_v, [lanes + 3])
                ids_v[pl.ds(g * 16, 16)] = ((v0 * 2 + v1) * 2 + v2) * 6 + v3

            def expand(t, carry):
                id_b = plsc.load_gather(ids_v, [jnp.full((16,), t, jnp.int32)])
                src = id_b * D + iota
                toff = t * D
                for j in range(D // 16):
                    v = plsc.load_gather(tab_v, [src + j * 16])
                    buf[pl.ds(toff + j * 16, 16)] = v
                return carry

            lax.fori_loop(0, C, expand, 0)
            pltpu.async_copy(buf, out_hbm.at[pl.ds(tok0 * D, C * D)], sem)

        def loop_body(i, carry):
            @pl.when(i % 2 == 0)
            def _():
                do_chunk(i, buf0, sem0)

            @pl.when(i % 2 == 1)
            def _():
                do_chunk(i, buf1, sem1)

            return carry

        lax.fori_loop(0, NCHUNK, loop_body, 0)
        pltpu.make_async_copy(buf0, out_hbm.at[pl.ds(base * D, C * D)], sem0).wait()
        pltpu.make_async_copy(buf1, out_hbm.at[pl.ds(base * D, C * D)], sem1).wait()

    return k(x_flat, fused)


def kernel(x, street_emb, action_emb, position_emb, order_emb):
    fused = _build_fused_table(street_emb, action_emb, position_emb, order_emb)
    x_flat = x.astype(jnp.int32).reshape(N * 4)
    out = _sc_lookup(x_flat, fused.reshape(V * D))
    return out.reshape(B, L, D)


# speedup vs baseline: 1.7979x; 1.7979x over previous
import functools

import jax
import jax.numpy as jnp
from jax import lax
from jax.experimental import pallas as pl
from jax.experimental.pallas import tpu as pltpu
from jax.experimental.pallas import tpu_sc as plsc

B, L, EMB = 16384, 200, 128
N = B * L
D = 4 * EMB
V = 4 * 2 * 2 * 6
NC, NS = 2, 16
NW = NC * NS
TOK_PER_W = N // NW
C = 64
NCHUNK = TOK_PER_W // C


def _build_fused_table(street, action, position, order):
    def body(s_ref, a_ref, p_ref, o_ref, out_ref):
        i = lax.broadcasted_iota(jnp.int32, (V, 1), 0)
        c0 = i // 24
        c1 = (i // 12) % 2
        c2 = (i // 6) % 2
        c3 = i % 6

        def sel(code, ref, nrows):
            out = jnp.broadcast_to(ref[nrows - 1:nrows, :], (V, EMB))
            for r in range(nrows - 2, -1, -1):
                row = jnp.broadcast_to(ref[r:r + 1, :], (V, EMB))
                out = jnp.where(code == r, row, out)
            return out

        out_ref[:, 0:EMB] = sel(c0, s_ref, 4)
        out_ref[:, EMB:2 * EMB] = sel(c1, p_ref, 2) + o_ref[0:1, :]
        out_ref[:, 2 * EMB:3 * EMB] = sel(c2, p_ref, 2) + o_ref[1:2, :]
        out_ref[:, 3 * EMB:4 * EMB] = sel(c3, a_ref, 6)

    return pl.pallas_call(
        body,
        out_shape=jax.ShapeDtypeStruct((V, D), jnp.float32),
    )(street, action, position, order)


def _sc_lookup(x_flat, fused):
    mesh = plsc.VectorSubcoreMesh(core_axis_name="c", subcore_axis_name="s")

    @functools.partial(
        pl.kernel,
        mesh=mesh,
        out_type=jax.ShapeDtypeStruct((N * D,), jnp.float32),
        compiler_params=pltpu.CompilerParams(needs_layout_passes=False),
        scratch_types=[
            pltpu.VMEM((C * 4,), jnp.int32),
            pltpu.VMEM((C,), jnp.int32),
            pltpu.VMEM((V * D,), jnp.float32),
            pltpu.VMEM((C * D,), jnp.float32),
            pltpu.VMEM((C * D,), jnp.float32),
            pltpu.SemaphoreType.DMA,
            pltpu.SemaphoreType.DMA,
        ],
    )
    def k(x_hbm, tab_hbm, out_hbm, x_v, ids_v, tab_v, buf0, buf1, sem0, sem1):
        cid = lax.axis_index("c")
        sid = lax.axis_index("s")
        wid = sid * NC + cid
        base = wid * TOK_PER_W

        iota = lax.iota(jnp.int32, 16)
        pltpu.sync_copy(tab_hbm, tab_v)

        def do_chunk(i, buf, sem):
            tok0 = base + i * C

            @pl.when(i >= 2)
            def _():
                pltpu.make_async_copy(
                    buf, out_hbm.at[pl.ds(tok0 * D, C * D)], sem).wait()

            pltpu.sync_copy(x_hbm.at[pl.ds(tok0 * 4, C * 4)], x_v)
            for g in range(C // 16):
                lanes = iota * 4 + g * 64
                v0 = plsc.load_gather(x_v, [lanes])
                v1 = plsc.load_gather(x_v, [lanes + 1])
                v2 = plsc.load_gather(x_v, [lanes + 2])
                v3 = plsc.load_gather(x_v, [lanes + 3])
                ids_v[pl.ds(g * 16, 16)] = ((v0 * 2 + v1) * 2 + v2) * 6 + v3

            @plsc.parallel_loop(0, C, unroll=2)
            def expand(t):
                id_b = plsc.load_gather(ids_v, [jnp.full((16,), t, jnp.int32)])
                src = id_b * D + iota
                toff = t * D
                for j in range(D // 16):
                    v = plsc.load_gather(tab_v, [src + j * 16])
                    buf[pl.ds(toff + j * 16, 16)] = v
            pltpu.async_copy(buf, out_hbm.at[pl.ds(tok0 * D, C * D)], sem)

        def loop_body(i, carry):
            @pl.when(i % 2 == 0)
            def _():
                do_chunk(i, buf0, sem0)

            @pl.when(i % 2 == 1)
            def _():
                do_chunk(i, buf1, sem1)

            return carry

        lax.fori_loop(0, NCHUNK, loop_body, 0)
        pltpu.make_async_copy(buf0, out_hbm.at[pl.ds(base * D, C * D)], sem0).wait()
        pltpu.make_async_copy(buf1, out_hbm.at[pl.ds(base * D, C * D)], sem1).wait()

    return k(x_flat, fused)


def kernel(x, street_emb, action_emb, position_emb, order_emb):
    fused = _build_fused_table(street_emb, action_emb, position_emb, order_emb)
    x_flat = x.astype(jnp.int32).reshape(N * 4)
    out = _sc_lookup(x_flat, fused.reshape(V * D))
    return out.reshape(B, L, D)
